# Initial kernel scaffold; baseline (speedup 1.0000x reference)
import jax
import jax.numpy as jnp
from jax.experimental import pallas as pl


def kernel(states_h, states_c, W_merge, b_merge, W_sum, b_sum):
    raise NotImplementedError("write your pallas kernel here")


# speedup vs baseline: 2.7253x; 2.7253x over previous
import jax
import jax.numpy as jnp
from jax.experimental import pallas as pl
from jax.experimental.pallas import tpu as pltpu

_D = 256
_T = 3072


def _lstm_merge(hcat, ccat, W, b):
    d = _D
    g = jnp.dot(hcat, W, preferred_element_type=jnp.float32) + b
    i = jax.nn.sigmoid(g[:, 0 * d:1 * d])
    o = jax.nn.sigmoid(g[:, 1 * d:2 * d])
    u = jnp.tanh(g[:, 2 * d:3 * d])
    fl = jax.nn.sigmoid(g[:, 3 * d:4 * d])
    fr = jax.nn.sigmoid(g[:, 4 * d:5 * d])
    c = i * u + fl * ccat[:, :d] + fr * ccat[:, d:]
    h = o * jnp.tanh(c)
    return h, c


def _fenwick_kernel(h_ref, c_ref, Wm_ref, bm_ref, Ws_ref, bs_ref,
                    ho_ref, co_ref):
    h = h_ref[...]
    c = c_ref[...]
    Wm = Wm_ref[...]
    bm = bm_ref[0]
    Ws = Ws_ref[...]
    bs = bs_ref[0]

    n = _T
    while n > 3:
        m = n // 2
        h, c = _lstm_merge(h.reshape(m, 2 * _D), c.reshape(m, 2 * _D),
                           Wm, bm)
        n = m

    hA, cA = _lstm_merge(h[0:2].reshape(1, 2 * _D),
                         c[0:2].reshape(1, 2 * _D), Wm, bm)
    hB = h[2:3]
    cB = c[2:3]
    hf, cf = _lstm_merge(jnp.concatenate([hB, hA], axis=1),
                         jnp.concatenate([cB, cA], axis=1), Ws, bs)
    ho_ref[...] = hf
    co_ref[...] = cf


def kernel(states_h, states_c, W_merge, b_merge, W_sum, b_sum):
    out_shape = (jax.ShapeDtypeStruct((1, _D), jnp.float32),
                 jax.ShapeDtypeStruct((1, _D), jnp.float32))
    h, c = pl.pallas_call(
        _fenwick_kernel,
        out_shape=out_shape,
    )(states_h, states_c, W_merge, b_merge.reshape(1, -1),
      W_sum, b_sum.reshape(1, -1))
    return (h, c)
